# Initial kernel scaffold; baseline (speedup 1.0000x reference)
import jax
import jax.numpy as jnp
from jax.experimental import pallas as pl


def kernel(index, content_emb, codebook_emb, frame_emb):
    raise NotImplementedError("write your pallas kernel here")


# speedup vs baseline: 3.6558x; 3.6558x over previous
import functools

import jax
import jax.numpy as jnp
from jax import lax
from jax.experimental import pallas as pl
from jax.experimental.pallas import tpu as pltpu
from jax.experimental.pallas import tpu_sc as plsc

NUM_CLASSES = 8192
B, K, L, D = 16, 8, 2048, 128
N = B * K * L

NC, NS, LANES = 2, 16, 16
NW = NC * NS
WPK = NW // K
BPW = B // WPK
C = 256
NCH = L // C
G = C // 128


def _body(idx_hbm, content_hbm, cb_hbm, fr_hbm, out_hbm,
          idx_v, rows_v, pos_v, cb_v, sem_g):
    wid = lax.axis_index("s") * NC + lax.axis_index("c")
    k = wid // WPK
    bg = wid % WPK

    pltpu.sync_copy(cb_hbm.at[k], cb_v)
    cbv = [cb_v[0, pl.ds(c * LANES, LANES)] for c in range(D // LANES)]

    for ch in range(NCH):
        l0 = ch * C
        pltpu.sync_copy(fr_hbm.at[pl.ds(l0, C)], pos_v)

        def addcb(r, carry):
            for c in range(D // LANES):
                plsc.addupdate(pos_v.at[r, pl.ds(c * LANES, LANES)], cbv[c])
            return carry
        lax.fori_loop(0, C, addcb, 0)

        for j in range(BPW):
            b = bg * BPW + j
            nbase = (b * K + k) * L + l0
            pltpu.sync_copy(idx_hbm.at[nbase // C], idx_v)
            cps = [pltpu.async_copy(content_hbm.at[idx_v.at[t]],
                                    rows_v.at[pl.ds(t * 128, 128)], sem_g)
                   for t in range(G)]
            for cp in cps:
                cp.wait()

            def addpos(r, carry):
                for c in range(D // LANES):
                    v = pos_v[r, pl.ds(c * LANES, LANES)]
                    plsc.addupdate(rows_v.at[r, pl.ds(c * LANES, LANES)], v)
                return carry
            lax.fori_loop(0, C, addpos, 0)

            pltpu.sync_copy(rows_v, out_hbm.at[pl.ds(pl.multiple_of(nbase, C), C)])


@jax.jit
def _run(idx2d, content_emb, codebook_emb, frame_emb):
    mesh = plsc.VectorSubcoreMesh(core_axis_name="c", subcore_axis_name="s")
    fn = pl.kernel(
        _body,
        out_type=jax.ShapeDtypeStruct((N, D), jnp.float32),
        mesh=mesh,
        scratch_types=[
            pltpu.VMEM((G, 128), jnp.int32),
            pltpu.VMEM((C, D), jnp.float32),
            pltpu.VMEM((C, D), jnp.float32),
            pltpu.VMEM((1, D), jnp.float32),
            pltpu.SemaphoreType.DMA,
        ],
    )
    return fn(idx2d, content_emb, codebook_emb, frame_emb)


def kernel(index, content_emb, codebook_emb, frame_emb):
    idx3d = index.reshape(N // C, G, 128)
    cb3d = codebook_emb.reshape(K, 1, D)
    out = _run(idx3d, content_emb, cb3d, frame_emb)
    return out.reshape(B, K, L, D)
